# baseline (speedup 1.0000x reference)
import functools

import jax
import jax.numpy as jnp
from jax import lax
from jax.experimental import pallas as pl
from jax.experimental.pallas import tpu as pltpu
from jax.experimental.pallas import tpu_sc as plsc

VOCAB = 100
HIDDEN = 128
BATCH = 4096
HIST = 50

_NC, _NS = 2, 16
_NW = _NC * _NS
_COLS_PW = BATCH // _NW
_TG = 1
_NGROUP = HIST // _TG
_RING = 6


def _make_gather():
    mesh = plsc.VectorSubcoreMesh(core_axis_name="c", subcore_axis_name="s")

    @functools.partial(
        pl.kernel,
        out_type=jax.ShapeDtypeStruct((HIST, BATCH, HIDDEN), jnp.float32),
        mesh=mesh,
        scratch_types=[
            pltpu.VMEM_SHARED((VOCAB, HIDDEN), jnp.float32),
            pltpu.VMEM((HIST, _COLS_PW), jnp.int32),
            *([pltpu.VMEM((_TG, _COLS_PW, HIDDEN), jnp.float32)] * _RING),
            *([pltpu.SemaphoreType.DMA] * (2 * _RING)),
        ],
    )
    def gather_kernel(idx_hbm, table_hbm, out_hbm, table_sh, idx_v, *rest):
        bufs = rest[:_RING]
        gsems = rest[_RING:2 * _RING]
        ssems = rest[2 * _RING:]
        sid = lax.axis_index("s")
        wid = sid * _NC + lax.axis_index("c")
        base = wid * _COLS_PW

        @pl.when(sid == 0)
        def _():
            pltpu.sync_copy(table_hbm, table_sh)

        pltpu.sync_copy(idx_hbm.at[:, pl.ds(base, _COLS_PW)], idx_v)
        plsc.subcore_barrier()

        def start_gathers(g, buf, sem, n=_TG):
            for b in range(n):
                pltpu.async_copy(
                    table_sh.at[idx_v.at[g * _TG + b]], buf.at[b], sem
                )

        def wait_gathers(g, buf, sem, n=_TG):
            for b in range(n):
                pltpu.make_async_copy(
                    table_sh.at[idx_v.at[g * _TG + b]], buf.at[b], sem
                ).wait()

        def start_store(g, buf, sem, n=_TG):
            pltpu.async_copy(
                buf.at[pl.ds(0, n)],
                out_hbm.at[pl.ds(g * _TG, n), pl.ds(base, _COLS_PW)],
                sem,
            )

        def wait_store(buf, sem, n=_TG):
            pltpu.make_async_copy(
                buf.at[pl.ds(0, n)],
                out_hbm.at[pl.ds(0, n), pl.ds(base, _COLS_PW)], sem
            ).wait()

        rings = tuple(zip(bufs, gsems, ssems))

        for k, (buf, gs, _) in enumerate(rings):
            start_gathers(k, buf, gs)

        def group_round(i):
            g = _RING * i
            for k, (buf, gs, ss) in enumerate(rings):
                wait_gathers(g + k, buf, gs)
                start_store(g + k, buf, ss)
            for k, (buf, gs, ss) in enumerate(rings):
                wait_store(buf, ss)
                start_gathers(g + _RING + k, buf, gs)

        _NFULL = (_NGROUP - _RING) // _RING
        pl.loop(0, _NFULL)(group_round)

        g0 = _NFULL * _RING
        for k, (buf, gs, ss) in enumerate(rings):
            wait_gathers(g0 + k, buf, gs)
            start_store(g0 + k, buf, ss)
        ntail = _NGROUP - (g0 + _RING)
        for k in range(ntail):
            buf, gs, ss = rings[k]
            wait_store(buf, ss)
            start_gathers(g0 + _RING + k, buf, gs)
        for k in range(ntail):
            buf, gs, ss = rings[k]
            wait_gathers(g0 + _RING + k, buf, gs)
            start_store(g0 + _RING + k, buf, ss)
        for buf, gs, ss in rings:
            wait_store(buf, ss)

    return gather_kernel


_gather = _make_gather()


def kernel(input_ids, word_embeddings):
    ids_t = input_ids.astype(jnp.int32).T
    out = _gather(ids_t, word_embeddings)
    return jnp.transpose(out, (1, 0, 2))
